# Initial kernel scaffold; baseline (speedup 1.0000x reference)
import functools

import jax
import jax.numpy as jnp
from jax import lax
from jax.experimental import pallas as pl
from jax.experimental.pallas import tpu as pltpu
from jax.experimental.pallas import tpu_sc as plsc

_G = 64
_NC = 2
_NS = 16
_NW = _NC * _NS


def _pick_chunk(epw):
    for cand in range(min(epw, 128), 0, -1):
        if epw % cand == 0 and (cand % 8 == 0 or cand < 8):
            return cand
    return 1


def _largest_div(n, cap):
    for cand in range(min(n, cap), 0, -1):
        if n % cand == 0:
            return cand
    return 1


@functools.lru_cache(maxsize=None)
def _build_agg(n, d, nchunk, ch):
    rows_per_tile = n // _NS
    zr = _largest_div(rows_per_tile, 32)
    nz = rows_per_tile // zr
    mesh = plsc.VectorSubcoreMesh(
        core_axis_name="c", subcore_axis_name="s",
        num_cores=_NC, num_subcores=_NS)

    def body(h_hbm, src_hbm, dst_hbm, out_hbm, src_v, dst_v, rows_v, zbuf,
             agg_sh, sem):
        c = lax.axis_index("c")
        s = lax.axis_index("s")
        wid = s * _NC + c
        zero = jnp.zeros((16,), jnp.float32)
        for i in range(zr):
            for j in range(d // 16):
                zbuf[i, pl.ds(j * 16, 16)] = zero
        base = s * rows_per_tile
        for k in range(nz):
            pltpu.sync_copy(zbuf, agg_sh.at[pl.ds(base + k * zr, zr)])
        plsc.subcore_barrier()
        pltpu.sync_copy(src_hbm.at[wid], src_v)
        pltpu.sync_copy(dst_hbm.at[wid], dst_v)

        def step(j, carry):
            pltpu.async_copy(h_hbm.at[src_v.at[j]], rows_v, sem).wait()
            pltpu.sync_copy(rows_v, agg_sh.at[dst_v.at[j]], add=True)
            return carry

        lax.fori_loop(0, nchunk, step, 0)
        plsc.subcore_barrier()
        pltpu.sync_copy(agg_sh.at[pl.ds(base, rows_per_tile)],
                        out_hbm.at[c, pl.ds(base, rows_per_tile)])

    return pl.kernel(
        body,
        out_type=jax.ShapeDtypeStruct((_NC, n, d), jnp.float32),
        mesh=mesh,
        scratch_types=[
            pltpu.VMEM((nchunk, ch), jnp.int32),
            pltpu.VMEM((nchunk, ch), jnp.int32),
            pltpu.VMEM((ch, d), jnp.float32),
            pltpu.VMEM((zr, d), jnp.float32),
            pltpu.VMEM_SHARED((n, d), jnp.float32),
            pltpu.SemaphoreType.DMA,
        ],
    )


def _mlp_body(h_ref, agg_ref, w1_ref, b1_ref, g_ref, be_ref, w2_ref,
              b2_ref, out_ref):
    z = h_ref[...] + agg_ref[0] + agg_ref[1]
    t = jnp.dot(z, w1_ref[...], preferred_element_type=jnp.float32)
    t = t + b1_ref[...]
    m = jnp.mean(t, axis=0, keepdims=True)
    v = jnp.mean(jnp.square(t - m), axis=0, keepdims=True)
    t = (t - m) * lax.rsqrt(v + 1e-5) * g_ref[...] + be_ref[...]
    t = jnp.maximum(t, 0.0)
    u = jnp.dot(t, w2_ref[...], preferred_element_type=jnp.float32)
    u = u + b2_ref[...]
    out_ref[...] = jnp.maximum(u, 0.0)


def _mlp_call(h, agg, w1, b1, g, be, w2, b2):
    n = h.shape[0]
    dout = w2.shape[1]
    return pl.pallas_call(
        _mlp_body,
        out_shape=jax.ShapeDtypeStruct((n, dout), jnp.float32),
    )(h, agg, w1, b1.reshape(1, -1), g.reshape(1, -1), be.reshape(1, -1),
      w2, b2.reshape(1, -1))


def _head_body(h_ref, seg_ref, l1w_ref, l1b_ref, l2w_ref, l2b_ref, out_ref):
    n = h_ref.shape[0]
    g_count = out_ref.shape[0]
    ids = lax.broadcasted_iota(jnp.int32, (g_count, n), 0)
    onehot = (ids == seg_ref[...]).astype(jnp.float32)
    p = jnp.dot(onehot, h_ref[...], preferred_element_type=jnp.float32)
    p = jnp.dot(p, l1w_ref[...], preferred_element_type=jnp.float32)
    p = jnp.maximum(p + l1b_ref[...], 0.0)
    p = jnp.dot(p, l2w_ref[...], preferred_element_type=jnp.float32)
    p = p + l2b_ref[...]
    mx = jnp.max(p, axis=1, keepdims=True)
    lse = mx + jnp.log(jnp.sum(jnp.exp(p - mx), axis=1, keepdims=True))
    out_ref[...] = p - lse


def _head_call(h, seg, l1w, l1b, l2w, l2b):
    n, d = h.shape
    dout = l2w.shape[1]
    l2w_p = jnp.pad(l2w, ((0, 0), (0, d - dout)))
    l2b_p = jnp.pad(l2b, (0, d - dout), constant_values=-1e30)
    out = pl.pallas_call(
        _head_body,
        out_shape=jax.ShapeDtypeStruct((_G, d), jnp.float32),
    )(h, seg.reshape(1, n), l1w, l1b.reshape(1, -1), l2w_p,
      l2b_p.reshape(1, -1))
    return out[:, :dout]


def kernel(x, edge_index, batch, params):
    n, d = x.shape
    e = edge_index.shape[1]
    epw = e // _NW
    ch = _pick_chunk(epw)
    nchunk = epw // ch

    src = edge_index[0].reshape(_NW, nchunk, ch)
    dst = edge_index[1].reshape(_NW, nchunk, ch)

    agg_fn = _build_agg(n, d, nchunk, ch)

    h = x
    for i in range(5):
        agg = agg_fn(h, src, dst)
        h = _mlp_call(h, agg, params['c%d_W1' % i], params['c%d_b1' % i],
                      params['c%d_g' % i], params['c%d_be' % i],
                      params['c%d_W2' % i], params['c%d_b2' % i])
    return _head_call(h, batch, params['lin1_W'], params['lin1_b'],
                      params['lin2_W'], params['lin2_b'])


# speedup vs baseline: 6.5497x; 6.5497x over previous
import functools

import jax
import jax.numpy as jnp
from jax import lax
from jax.experimental import pallas as pl
from jax.experimental.pallas import tpu as pltpu
from jax.experimental.pallas import tpu_sc as plsc

_G = 64
_NC = 2
_NS = 16
_NW = _NC * _NS


def _pick_chunk(epw):
    for cand in range(min(epw, 128), 0, -1):
        if epw % cand == 0 and (cand % 8 == 0 or cand < 8):
            return cand
    return 1


def _largest_div(n, cap):
    for cand in range(min(n, cap), 0, -1):
        if n % cand == 0:
            return cand
    return 1


@functools.lru_cache(maxsize=None)
def _build_agg(n, d, nchunk, ch):
    align = _NS * 32
    n_pad = (n + align - 1) // align * align
    rows_per_tile = n_pad // _NS
    zr = 32
    nz = rows_per_tile // zr
    mesh = plsc.VectorSubcoreMesh(
        core_axis_name="c", subcore_axis_name="s",
        num_cores=_NC, num_subcores=_NS)

    def body(h_hbm, src_hbm, dst_hbm, out_hbm, src_v, dst_v, rows_v, zbuf,
             agg_sh, sem):
        c = lax.axis_index("c")
        s = lax.axis_index("s")
        wid = s * _NC + c
        zero = jnp.zeros((16,), jnp.float32)
        for i in range(zr):
            for j in range(d // 16):
                zbuf[i, pl.ds(j * 16, 16)] = zero
        base = s * rows_per_tile
        for k in range(nz):
            pltpu.sync_copy(zbuf, agg_sh.at[pl.ds(base + k * zr, zr)])
        plsc.subcore_barrier()
        pltpu.sync_copy(src_hbm.at[wid], src_v)
        pltpu.sync_copy(dst_hbm.at[wid], dst_v)

        def step(j, carry):
            pltpu.async_copy(h_hbm.at[src_v.at[j]], rows_v, sem).wait()
            pltpu.sync_copy(rows_v, agg_sh.at[dst_v.at[j]], add=True)
            return carry

        lax.fori_loop(0, nchunk, step, 0)
        plsc.subcore_barrier()
        pltpu.sync_copy(agg_sh.at[pl.ds(base, rows_per_tile)],
                        out_hbm.at[c, pl.ds(base, rows_per_tile)])

    return pl.kernel(
        body,
        out_type=jax.ShapeDtypeStruct((_NC, n_pad, d), jnp.float32),
        mesh=mesh,
        scratch_types=[
            pltpu.VMEM((nchunk, ch), jnp.int32),
            pltpu.VMEM((nchunk, ch), jnp.int32),
            pltpu.VMEM((ch, d), jnp.float32),
            pltpu.VMEM((zr, d), jnp.float32),
            pltpu.VMEM_SHARED((n_pad, d), jnp.float32),
            pltpu.SemaphoreType.DMA,
        ],
    )


def _mlp_body(h_ref, agg_ref, w1_ref, b1_ref, g_ref, be_ref, w2_ref,
              b2_ref, out_ref):
    n = h_ref.shape[0]
    z = h_ref[...] + agg_ref[0, :n] + agg_ref[1, :n]
    t = jnp.dot(z, w1_ref[...], preferred_element_type=jnp.float32)
    t = t + b1_ref[...]
    m = jnp.mean(t, axis=0, keepdims=True)
    v = jnp.mean(jnp.square(t - m), axis=0, keepdims=True)
    t = (t - m) * lax.rsqrt(v + 1e-5) * g_ref[...] + be_ref[...]
    t = jnp.maximum(t, 0.0)
    u = jnp.dot(t, w2_ref[...], preferred_element_type=jnp.float32)
    u = u + b2_ref[...]
    out_ref[...] = jnp.maximum(u, 0.0)


def _mlp_call(h, agg, w1, b1, g, be, w2, b2):
    n = h.shape[0]
    dout = w2.shape[1]
    return pl.pallas_call(
        _mlp_body,
        out_shape=jax.ShapeDtypeStruct((n, dout), jnp.float32),
    )(h, agg, w1, b1.reshape(1, -1), g.reshape(1, -1), be.reshape(1, -1),
      w2, b2.reshape(1, -1))


def _head_body(h_ref, seg_ref, l1w_ref, l1b_ref, l2w_ref, l2b_ref, out_ref):
    n = h_ref.shape[0]
    g_count = out_ref.shape[0]
    ids = lax.broadcasted_iota(jnp.int32, (g_count, n), 0)
    onehot = (ids == seg_ref[...]).astype(jnp.float32)
    p = jnp.dot(onehot, h_ref[...], preferred_element_type=jnp.float32)
    p = jnp.dot(p, l1w_ref[...], preferred_element_type=jnp.float32)
    p = jnp.maximum(p + l1b_ref[...], 0.0)
    p = jnp.dot(p, l2w_ref[...], preferred_element_type=jnp.float32)
    p = p + l2b_ref[...]
    mx = jnp.max(p, axis=1, keepdims=True)
    lse = mx + jnp.log(jnp.sum(jnp.exp(p - mx), axis=1, keepdims=True))
    out_ref[...] = p - lse


def _head_call(h, seg, l1w, l1b, l2w, l2b):
    n, d = h.shape
    dout = l2w.shape[1]
    l2w_p = jnp.pad(l2w, ((0, 0), (0, d - dout)))
    l2b_p = jnp.pad(l2b, (0, d - dout), constant_values=-1e30)
    out = pl.pallas_call(
        _head_body,
        out_shape=jax.ShapeDtypeStruct((_G, d), jnp.float32),
    )(h, seg.reshape(1, n), l1w, l1b.reshape(1, -1), l2w_p,
      l2b_p.reshape(1, -1))
    return out[:, :dout]


def kernel(x, edge_index, batch, params):
    n, d = x.shape
    e = edge_index.shape[1]
    epw = e // _NW
    ch = _pick_chunk(epw)
    nchunk = epw // ch

    src = edge_index[0].reshape(_NW, nchunk, ch)
    dst = edge_index[1].reshape(_NW, nchunk, ch)

    agg_fn = _build_agg(n, d, nchunk, ch)

    h = x
    for i in range(5):
        agg = agg_fn(h, src, dst)
        h = _mlp_call(h, agg, params['c%d_W1' % i], params['c%d_b1' % i],
                      params['c%d_g' % i], params['c%d_be' % i],
                      params['c%d_W2' % i], params['c%d_b2' % i])
    return _head_call(h, batch, params['lin1_W'], params['lin1_b'],
                      params['lin2_W'], params['lin2_b'])


# speedup vs baseline: 10.7701x; 1.6444x over previous
import functools

import jax
import jax.numpy as jnp
from jax import lax
from jax.experimental import pallas as pl
from jax.experimental.pallas import tpu as pltpu
from jax.experimental.pallas import tpu_sc as plsc

_G = 64
_NC = 2
_NS = 16
_NW = _NC * _NS


def _pick_chunk(epw):
    for cand in range(min(epw, 128), 0, -1):
        if epw % cand == 0 and (cand % 8 == 0 or cand < 8):
            return cand
    return 1


def _largest_div(n, cap):
    for cand in range(min(n, cap), 0, -1):
        if n % cand == 0:
            return cand
    return 1


@functools.lru_cache(maxsize=None)
def _build_agg(n, d, nchunk, ch):
    align = _NS * 32
    n_pad = (n + align - 1) // align * align
    rows_per_tile = n_pad // _NS
    zr = 32
    nz = rows_per_tile // zr
    assert nchunk % 2 == 1, "pipeline below assumes an odd chunk count"
    assert ch >= zr, "rows_a doubles as the zero source"
    mesh = plsc.VectorSubcoreMesh(
        core_axis_name="c", subcore_axis_name="s",
        num_cores=_NC, num_subcores=_NS)

    npairs = (nchunk - 1) // 2

    def body(h_hbm, src_hbm, dst_hbm, out_hbm, src_v, rows_a, rows_b,
             dst_a, dst_b, agg_sh, sem_a, sem_b, sem_da, sem_db):
        c = lax.axis_index("c")
        s = lax.axis_index("s")
        wid = s * _NC + c
        zero = jnp.zeros((16,), jnp.float32)
        for i in range(zr):
            for j in range(d // 16):
                rows_a[i, pl.ds(j * 16, 16)] = zero
        base = s * rows_per_tile
        for k in range(nz):
            pltpu.sync_copy(rows_a.at[pl.ds(0, zr)],
                            agg_sh.at[pl.ds(base + k * zr, zr)])
        pltpu.sync_copy(src_hbm.at[wid], src_v)
        pltpu.async_copy(dst_hbm.at[wid, 0], dst_a, sem_da)
        pltpu.async_copy(h_hbm.at[src_v.at[0]], rows_a, sem_a)
        plsc.subcore_barrier()

        def step(k, carry):
            a = 2 * k
            pltpu.async_copy(dst_hbm.at[wid, a + 1], dst_b, sem_db)
            pltpu.async_copy(h_hbm.at[src_v.at[a + 1]], rows_b, sem_b)
            pltpu.make_async_copy(h_hbm.at[src_v.at[a]], rows_a, sem_a).wait()
            pltpu.make_async_copy(dst_hbm.at[wid, a], dst_a, sem_da).wait()
            pltpu.sync_copy(rows_a, agg_sh.at[dst_a.at[0]], add=True)
            pltpu.async_copy(dst_hbm.at[wid, a + 2], dst_a, sem_da)
            pltpu.async_copy(h_hbm.at[src_v.at[a + 2]], rows_a, sem_a)
            pltpu.make_async_copy(
                h_hbm.at[src_v.at[a + 1]], rows_b, sem_b).wait()
            pltpu.make_async_copy(
                dst_hbm.at[wid, a + 1], dst_b, sem_db).wait()
            pltpu.sync_copy(rows_b, agg_sh.at[dst_b.at[0]], add=True)
            return carry

        lax.fori_loop(0, npairs, step, 0)
        last = nchunk - 1
        pltpu.make_async_copy(h_hbm.at[src_v.at[last]], rows_a, sem_a).wait()
        pltpu.make_async_copy(dst_hbm.at[wid, last], dst_a, sem_da).wait()
        pltpu.sync_copy(rows_a, agg_sh.at[dst_a.at[0]], add=True)
        plsc.subcore_barrier()
        pltpu.sync_copy(agg_sh.at[pl.ds(base, rows_per_tile)],
                        out_hbm.at[c, pl.ds(base, rows_per_tile)])

    return pl.kernel(
        body,
        out_type=jax.ShapeDtypeStruct((_NC, n_pad, d), jnp.float32),
        mesh=mesh,
        scratch_types=[
            pltpu.VMEM((nchunk, ch), jnp.int32),
            pltpu.VMEM((ch, d), jnp.float32),
            pltpu.VMEM((ch, d), jnp.float32),
            pltpu.VMEM((1, ch), jnp.int32),
            pltpu.VMEM((1, ch), jnp.int32),
            pltpu.VMEM_SHARED((n_pad, d), jnp.float32),
            pltpu.SemaphoreType.DMA,
            pltpu.SemaphoreType.DMA,
            pltpu.SemaphoreType.DMA,
            pltpu.SemaphoreType.DMA,
        ],
    )


def _mlp_body(h_ref, agg_ref, w1_ref, b1_ref, g_ref, be_ref, w2_ref,
              b2_ref, out_ref):
    n = h_ref.shape[0]
    z = h_ref[...] + agg_ref[0, :n] + agg_ref[1, :n]
    t = jnp.dot(z, w1_ref[...], preferred_element_type=jnp.float32)
    t = t + b1_ref[...]
    m = jnp.mean(t, axis=0, keepdims=True)
    v = jnp.mean(jnp.square(t - m), axis=0, keepdims=True)
    t = (t - m) * lax.rsqrt(v + 1e-5) * g_ref[...] + be_ref[...]
    t = jnp.maximum(t, 0.0)
    u = jnp.dot(t, w2_ref[...], preferred_element_type=jnp.float32)
    u = u + b2_ref[...]
    out_ref[...] = jnp.maximum(u, 0.0)


def _mlp_call(h, agg, w1, b1, g, be, w2, b2):
    n = h.shape[0]
    dout = w2.shape[1]
    return pl.pallas_call(
        _mlp_body,
        out_shape=jax.ShapeDtypeStruct((n, dout), jnp.float32),
    )(h, agg, w1, b1.reshape(1, -1), g.reshape(1, -1), be.reshape(1, -1),
      w2, b2.reshape(1, -1))


def _head_body(h_ref, seg_ref, l1w_ref, l1b_ref, l2w_ref, l2b_ref, out_ref):
    n = h_ref.shape[0]
    g_count = out_ref.shape[0]
    ids = lax.broadcasted_iota(jnp.int32, (g_count, n), 0)
    onehot = (ids == seg_ref[...]).astype(jnp.float32)
    p = jnp.dot(onehot, h_ref[...], preferred_element_type=jnp.float32)
    p = jnp.dot(p, l1w_ref[...], preferred_element_type=jnp.float32)
    p = jnp.maximum(p + l1b_ref[...], 0.0)
    p = jnp.dot(p, l2w_ref[...], preferred_element_type=jnp.float32)
    p = p + l2b_ref[...]
    mx = jnp.max(p, axis=1, keepdims=True)
    lse = mx + jnp.log(jnp.sum(jnp.exp(p - mx), axis=1, keepdims=True))
    out_ref[...] = p - lse


def _head_call(h, seg, l1w, l1b, l2w, l2b):
    n, d = h.shape
    dout = l2w.shape[1]
    l2w_p = jnp.pad(l2w, ((0, 0), (0, d - dout)))
    l2b_p = jnp.pad(l2b, (0, d - dout), constant_values=-1e30)
    out = pl.pallas_call(
        _head_body,
        out_shape=jax.ShapeDtypeStruct((_G, d), jnp.float32),
    )(h, seg.reshape(1, n), l1w, l1b.reshape(1, -1), l2w_p,
      l2b_p.reshape(1, -1))
    return out[:, :dout]


def kernel(x, edge_index, batch, params):
    n, d = x.shape
    e = edge_index.shape[1]
    epw = e // _NW
    ch = _pick_chunk(epw)
    nchunk = epw // ch

    src = edge_index[0].reshape(_NW, nchunk, ch)
    dst = edge_index[1].reshape(_NW, nchunk, 1, ch)

    agg_fn = _build_agg(n, d, nchunk, ch)

    h = x
    for i in range(5):
        agg = agg_fn(h, src, dst)
        h = _mlp_call(h, agg, params['c%d_W1' % i], params['c%d_b1' % i],
                      params['c%d_g' % i], params['c%d_be' % i],
                      params['c%d_W2' % i], params['c%d_b2' % i])
    return _head_call(h, batch, params['lin1_W'], params['lin1_b'],
                      params['lin2_W'], params['lin2_b'])
